# baseline (speedup 1.0000x reference)
import jax
import jax.numpy as jnp
from jax import lax
from jax.experimental import pallas as pl
from jax.experimental.pallas import tpu as pltpu
from jax.experimental.pallas import tpu_sc as plsc

N_NODES = 10000
D_FEAT = 256
N_EDGES = 160000

NUM_CORES = 2
NUM_SUBCORES = 16
NUM_WORKERS = NUM_CORES * NUM_SUBCORES
CHUNK = 96
NUM_CHUNKS = 54
NUM_PAIRS = NUM_CHUNKS // 2
EDGES_PER_WORKER = CHUNK * NUM_CHUNKS
E_PAD = EDGES_PER_WORKER * NUM_WORKERS
LANES = 16


def _sc_body(emb_hbm, src_hbm, dst_hbm, out_hbm,
             idx_s_v, idx_t_v, rows_s_v, rows_t_v, rows_s1, rows_t1, out_v,
             stage_flat_v, sem, sem1):
    wid = lax.axis_index("s") * NUM_CORES + lax.axis_index("c")
    base = wid * EDGES_PER_WORKER

    pltpu.sync_copy(src_hbm.at[pl.ds(base, EDGES_PER_WORKER)], idx_s_v)
    pltpu.sync_copy(dst_hbm.at[pl.ds(base, EDGES_PER_WORKER)], idx_t_v)

    def fire(ci, rows_s, rows_t, s):
        off = ci * CHUNK
        pltpu.async_copy(emb_hbm.at[idx_s_v.at[pl.ds(off, CHUNK)]], rows_s, s)
        pltpu.async_copy(emb_hbm.at[idx_t_v.at[pl.ds(off, CHUNK)]], rows_t, s)

    def drain(ci, rows_s, rows_t, s):
        off = ci * CHUNK
        pltpu.make_async_copy(
            emb_hbm.at[idx_s_v.at[pl.ds(off, CHUNK)]], rows_s, s).wait()
        pltpu.make_async_copy(
            emb_hbm.at[idx_t_v.at[pl.ds(off, CHUNK)]], rows_t, s).wait()

    lane = lax.iota(jnp.int32, LANES)
    gather_base = lane * LANES

    def compute(ci, rows_s, rows_t):
        off = ci * CHUNK

        def group_body(g, carry2):
            hi_mask = jnp.int32(-65536)
            e0 = g * LANES
            for k in range(LANES):
                e = e0 + k
                accs = [None, None, None, None]
                for j in range(D_FEAT // (2 * LANES)):
                    s_bf = plsc.bitcast(rows_s[e, pl.ds(j * LANES, LANES)],
                                        jnp.bfloat16)
                    t_bf = plsc.bitcast(rows_t[e, pl.ds(j * LANES, LANES)],
                                        jnp.bfloat16)
                    p_i = plsc.bitcast(s_bf * t_bf, jnp.int32)
                    a = plsc.bitcast(p_i << 16, jnp.float32)
                    b = plsc.bitcast(p_i & hi_mask, jnp.float32)
                    kk = j & 3
                    accs[kk] = a + b if accs[kk] is None else accs[kk] + (a + b)
                stage_flat_v[pl.ds(k * LANES, LANES)] = (
                    (accs[0] + accs[1]) + (accs[2] + accs[3]))
            tots = None
            for j in range(LANES):
                col = plsc.load_gather(stage_flat_v, [gather_base + j])
                tots = col if tots is None else tots + col
            out_v[pl.ds(off + e0, LANES)] = tots
            return carry2

        lax.fori_loop(0, CHUNK // LANES, group_body, 0)

    fire(0, rows_s_v, rows_t_v, sem)

    def pair_body(p, carry):
        c0 = 2 * p
        fire(c0 + 1, rows_s1, rows_t1, sem1)
        drain(c0, rows_s_v, rows_t_v, sem)
        compute(c0, rows_s_v, rows_t_v)

        @pl.when(p < NUM_PAIRS - 1)
        def _():
            fire(c0 + 2, rows_s_v, rows_t_v, sem)

        drain(c0 + 1, rows_s1, rows_t1, sem1)
        compute(c0 + 1, rows_s1, rows_t1)
        return carry

    lax.fori_loop(0, NUM_PAIRS, pair_body, 0)
    pltpu.sync_copy(out_v, out_hbm.at[pl.ds(base, EDGES_PER_WORKER)])


def kernel(node_embeddings, edge_label_index):
    idx = edge_label_index.astype(jnp.int32)
    pad1 = (jnp.arange(E_PAD - N_EDGES, dtype=jnp.int32) * 13) % N_NODES
    pad = jnp.stack([pad1, pad1])
    idx = jnp.concatenate([idx, pad], axis=1)
    src = idx[0]
    dst = idx[1]

    mesh = plsc.VectorSubcoreMesh(core_axis_name="c", subcore_axis_name="s")
    f = pl.kernel(
        _sc_body,
        mesh=mesh,
        compiler_params=pltpu.CompilerParams(needs_layout_passes=False),
        out_type=jax.ShapeDtypeStruct((E_PAD,), jnp.float32),
        scratch_types=[
            pltpu.VMEM((EDGES_PER_WORKER,), jnp.int32),
            pltpu.VMEM((EDGES_PER_WORKER,), jnp.int32),
            pltpu.VMEM((CHUNK, D_FEAT // 2), jnp.int32),
            pltpu.VMEM((CHUNK, D_FEAT // 2), jnp.int32),
            pltpu.VMEM((CHUNK, D_FEAT // 2), jnp.int32),
            pltpu.VMEM((CHUNK, D_FEAT // 2), jnp.int32),
            pltpu.VMEM((EDGES_PER_WORKER,), jnp.float32),
            pltpu.VMEM((LANES * LANES,), jnp.float32),
            pltpu.SemaphoreType.DMA,
            pltpu.SemaphoreType.DMA,
        ],
    )
    u = jax.lax.bitcast_convert_type(node_embeddings, jnp.uint32)

    def round_bf16(x):
        return (x + jnp.uint32(0x7FFF) + ((x >> 16) & jnp.uint32(1))) >> 16

    hi = round_bf16(u[:, D_FEAT // 2:])
    lo = round_bf16(u[:, :D_FEAT // 2])
    packed = (hi << 16) | lo
    emb_i32 = jax.lax.bitcast_convert_type(packed, jnp.int32)
    return f(emb_i32, src, dst)[:N_EDGES]
